# Initial kernel scaffold; baseline (speedup 1.0000x reference)
import jax
import jax.numpy as jnp
from jax.experimental import pallas as pl


def kernel(user, user_hist, user_nbrs, pos_item, neg_item, params):
    raise NotImplementedError("write your pallas kernel here")


# speedup vs baseline: 1.1012x; 1.1012x over previous
import functools

import jax
import jax.numpy as jnp
from jax import lax
from jax.experimental import pallas as pl
from jax.experimental.pallas import tpu as pltpu
from jax.experimental.pallas import tpu_sc as plsc

B = 4096
HIST = 200
NBRS = 50
EDIM = 32
HISTP = 224
NBRSP = 64
RH = HISTP // 4
RN = NBRSP // 4

NW = 32
SUB = 128
GRP = 4
CH = SUB * GRP

HIST_PW = B * HISTP // NW
NBR_PW = B * NBRSP // NW
ONE_PW = B // NW

BB = 128


def _sc_gather_body(item_t, user_t, hist_i, nbr_i, user_i, pos_i, neg_i,
                    hist_o, nbr_o, user_o, pos_o, neg_o,
                    hist_iv, nbr_iv, one_iv, rows_v, sem):
    c = lax.axis_index("c")
    s = lax.axis_index("s")
    wid = s * 2 + c

    pltpu.sync_copy(hist_i.at[wid], hist_iv)
    pltpu.sync_copy(nbr_i.at[wid], nbr_iv)

    def grp(table, idx_v, out, base_rows, g):
        cps = [pltpu.async_copy(table.at[idx_v.at[g * GRP + j]],
                                rows_v.at[pl.ds(j * SUB, SUB)], sem)
               for j in range(GRP)]
        for cp in cps:
            cp.wait()
        pltpu.sync_copy(rows_v, out.at[pl.ds(base_rows + g * CH, CH)])

    def hist_body(g, carry):
        grp(item_t, hist_iv, hist_o, wid * HIST_PW, g)
        return carry

    lax.fori_loop(0, HIST_PW // CH, hist_body, 0)

    def nbr_body(g, carry):
        grp(user_t, nbr_iv, nbr_o, wid * NBR_PW, g)
        return carry

    lax.fori_loop(0, NBR_PW // CH, nbr_body, 0)

    for idx_hbm, table, out in ((user_i, user_t, user_o),
                                (pos_i, item_t, pos_o),
                                (neg_i, item_t, neg_o)):
        pltpu.sync_copy(idx_hbm.at[wid], one_iv)
        pltpu.async_copy(table.at[one_iv.at[0]],
                         rows_v.at[pl.ds(0, SUB)], sem).wait()
        pltpu.sync_copy(rows_v.at[pl.ds(0, SUB)],
                        out.at[pl.ds(wid * ONE_PW, ONE_PW)])


@functools.cache
def _sc_gather():
    return functools.partial(
        pl.kernel,
        out_type=[
            jax.ShapeDtypeStruct((B * HISTP, EDIM), jnp.float32),
            jax.ShapeDtypeStruct((B * NBRSP, EDIM), jnp.float32),
            jax.ShapeDtypeStruct((B, EDIM), jnp.float32),
            jax.ShapeDtypeStruct((B, EDIM), jnp.float32),
            jax.ShapeDtypeStruct((B, EDIM), jnp.float32),
        ],
        mesh=plsc.VectorSubcoreMesh(core_axis_name="c", subcore_axis_name="s"),
        compiler_params=pltpu.CompilerParams(use_tc_tiling_on_sc=False),
        scratch_types=[
            pltpu.VMEM((HIST_PW // SUB, SUB), jnp.int32),
            pltpu.VMEM((NBR_PW // SUB, SUB), jnp.int32),
            pltpu.VMEM((1, SUB), jnp.int32),
            pltpu.VMEM((CH, EDIM), jnp.float32),
            pltpu.SemaphoreType.DMA,
        ],
    )(_sc_gather_body)


def _tc_forward_body(hist_ref, histidx_ref, nbr_ref, nbridx_ref,
                     u_ref, pos_ref, neg_ref,
                     ia_wblk, ia_wu, ia_b1, ia_w2sel,
                     ua_wblk, ua_wu, ua_b1, ua_w2sel,
                     rep, scat,
                     fusew, fuseb, selfw, selfb, ul1w, ul1b, ul2w, ul2b,
                     il1w, il1b, il2w, il2b, rp1w, rp1b, rp2w, rp2b,
                     rp3wt, rp3b, pos_out, neg_out):
    u = u_ref[...]

    def attn(xp3, idx4, wblk, wu, b1, w2sel, R):
        pre = jnp.dot(u, wu) + b1
        pre128 = jnp.concatenate([pre] * 4, axis=1)
        t2 = jnp.dot(xp3.reshape(BB * R, 128), wblk)
        h3 = jnp.maximum(t2.reshape(BB, R, 128) + pre128[:, None, :], 0.0)
        s4 = jnp.dot(h3.reshape(BB * R, 128), w2sel).reshape(BB, R, 4)
        e4 = jnp.exp(s4) * (idx4 != 0).astype(jnp.float32)
        z = jnp.sum(jnp.sum(e4, axis=1), axis=1, keepdims=True)
        a128 = jnp.dot(e4.reshape(BB * R, 4), rep[...]).reshape(BB, R, 128)
        w = jnp.sum(a128 * xp3, axis=1)
        return jnp.dot(w, scat[...]) / jnp.maximum(z, 1e-35)

    h_item = attn(hist_ref[...], histidx_ref[...], ia_wblk[...],
                  ia_wu[...], ia_b1[...], ia_w2sel[...], RH)
    h_soc = attn(nbr_ref[...], nbridx_ref[...], ua_wblk[...],
                 ua_wu[...], ua_b1[...], ua_w2sel[...], RN)

    h = jnp.maximum(
        jnp.dot(jnp.concatenate([h_item, h_soc], axis=1), fusew[...]) + fuseb[...],
        0.0)
    hu = jnp.dot(jnp.concatenate([h, u], axis=1), selfw[...]) + selfb[...]
    hu = jnp.dot(jnp.maximum(jnp.dot(hu, ul1w[...]) + ul1b[...], 0.0),
                 ul2w[...]) + ul2b[...]

    def item_mlp(x):
        return jnp.dot(jnp.maximum(jnp.dot(x, il1w[...]) + il1b[...], 0.0),
                       il2w[...]) + il2b[...]

    ph = item_mlp(pos_ref[...])
    nh = item_mlp(neg_ref[...])

    def rp(x):
        x = jnp.maximum(jnp.dot(x, rp1w[...]) + rp1b[...], 0.0)
        x = jnp.maximum(jnp.dot(x, rp2w[...]) + rp2b[...], 0.0)
        return jnp.sum(x * rp3wt[...], axis=1, keepdims=True) + rp3b[0, 0]

    pos_out[...] = rp(jnp.concatenate([hu, ph], axis=1))
    neg_out[...] = rp(jnp.concatenate([hu, nh], axis=1))


def _tc_forward(histp, histidx4, nbrp, nbridx4, u_rows, pos_rows, neg_rows,
                weights):
    grid = (B // BB,)
    data_specs = [
        pl.BlockSpec((BB, RH, 128), lambda i: (i, 0, 0)),
        pl.BlockSpec((BB, RH, 4), lambda i: (i, 0, 0)),
        pl.BlockSpec((BB, RN, 128), lambda i: (i, 0, 0)),
        pl.BlockSpec((BB, RN, 4), lambda i: (i, 0, 0)),
        pl.BlockSpec((BB, EDIM), lambda i: (i, 0)),
        pl.BlockSpec((BB, EDIM), lambda i: (i, 0)),
        pl.BlockSpec((BB, EDIM), lambda i: (i, 0)),
    ]
    w_specs = [pl.BlockSpec(w.shape, lambda i: (0,) * w.ndim) for w in weights]
    out_specs = [pl.BlockSpec((BB, 1), lambda i: (i, 0))] * 2
    return pl.pallas_call(
        _tc_forward_body,
        grid=grid,
        in_specs=data_specs + w_specs,
        out_specs=out_specs,
        out_shape=[jax.ShapeDtypeStruct((B, 1), jnp.float32)] * 2,
    )(histp, histidx4, nbrp, nbridx4, u_rows, pos_rows, neg_rows, *weights)


def kernel(user, user_hist, user_nbrs, pos_item, neg_item, params):
    item_t = params['item_embs']
    user_t = params['user_embs']

    hist_pad = jnp.pad(user_hist, ((0, 0), (0, HISTP - HIST)))
    nbr_pad = jnp.pad(user_nbrs, ((0, 0), (0, NBRSP - NBRS)))

    hist_i = hist_pad.reshape(NW, HIST_PW // SUB, SUB)
    nbr_i = nbr_pad.reshape(NW, NBR_PW // SUB, SUB)
    user_i = user.reshape(NW, 1, SUB)
    pos_i = pos_item.reshape(NW, 1, SUB)
    neg_i = neg_item.reshape(NW, 1, SUB)

    hist_rows, nbr_rows, u_rows, pos_rows, neg_rows = _sc_gather()(
        item_t, user_t, hist_i, nbr_i, user_i, pos_i, neg_i)

    histp = hist_rows.reshape(B, RH, 128)
    nbrp = nbr_rows.reshape(B, RN, 128)
    histidx4 = hist_pad.reshape(B, RH, 4)
    nbridx4 = nbr_pad.reshape(B, RN, 4)

    p = params
    eye4 = jnp.eye(4, dtype=jnp.float32)

    def row(b):
        return b.reshape(1, -1)

    def attn_weights(p1, p2):
        w1, b1 = p1
        w2, _ = p2
        wblk = jnp.kron(eye4, w1[:EDIM, :])
        wu = w1[EDIM:, :]
        w2sel = jnp.kron(eye4, w2)
        return wblk, wu, row(b1), w2sel

    rep = jnp.kron(eye4, jnp.ones((1, EDIM), jnp.float32))
    scat = jnp.kron(jnp.ones((4, 1), jnp.float32),
                    jnp.eye(EDIM, dtype=jnp.float32))

    weights = (
        *attn_weights(p['ia1'], p['ia2']),
        *attn_weights(p['ua1'], p['ua2']),
        rep, scat,
        p['fuse'][0], row(p['fuse'][1]), p['self'][0], row(p['self'][1]),
        p['ul1'][0], row(p['ul1'][1]), p['ul2'][0], row(p['ul2'][1]),
        p['il1'][0], row(p['il1'][1]), p['il2'][0], row(p['il2'][1]),
        p['rp1'][0], row(p['rp1'][1]), p['rp2'][0], row(p['rp2'][1]),
        p['rp3'][0].T, p['rp3'][1].reshape(1, 1),
    )

    return _tc_forward(histp, histidx4, nbrp, nbridx4,
                       u_rows, pos_rows, neg_rows, weights)


# speedup vs baseline: 1.1095x; 1.0075x over previous
import functools

import jax
import jax.numpy as jnp
from jax import lax
from jax.experimental import pallas as pl
from jax.experimental.pallas import tpu as pltpu
from jax.experimental.pallas import tpu_sc as plsc

B = 4096
HIST = 200
NBRS = 50
EDIM = 32
HISTP = 224
NBRSP = 64
RH = HISTP // 4
RN = NBRSP // 4

NW = 32
SUB = 128
K = 8
CHB = SUB * K

HIST_PW = B * HISTP // NW
NBR_PW = B * NBRSP // NW
ONE_PW = B // NW

BB = 128


def _sc_gather_body(item_t, user_t, hist_i, nbr_i, user_i, pos_i, neg_i,
                    hist_o, nbr_o, user_o, pos_o, neg_o,
                    hist_iv, nbr_iv, one_iv, rows0, rows1,
                    gs0, gs1, ws0, ws1):
    c = lax.axis_index("c")
    s = lax.axis_index("s")
    wid = s * 2 + c

    pltpu.sync_copy(hist_i.at[wid], hist_iv)
    pltpu.sync_copy(nbr_i.at[wid], nbr_iv)

    def phase(table, idx_v, out, base, ngrp):
        def issue(g, buf, sem):
            for j in range(K):
                pltpu.async_copy(table.at[idx_v.at[g * K + j]],
                                 buf.at[pl.ds(j * SUB, SUB)], sem)

        def gwait(buf, sem):
            pltpu.make_async_copy(table.at[pl.ds(0, CHB)], buf, sem).wait()

        def wstart(g, buf, sem):
            pltpu.async_copy(buf, out.at[pl.ds(base + g * CHB, CHB)], sem)

        def wwait(buf, sem):
            pltpu.make_async_copy(buf, out.at[pl.ds(base, CHB)], sem).wait()

        npair = ngrp // 2
        issue(0, rows0, gs0)
        issue(1, rows1, gs1)

        def body(t, carry):
            g = 2 * t
            gwait(rows0, gs0)
            wstart(g, rows0, ws0)
            gwait(rows1, gs1)
            wstart(g + 1, rows1, ws1)

            @pl.when(t + 1 < npair)
            def _():
                wwait(rows0, ws0)
                issue(g + 2, rows0, gs0)
                wwait(rows1, ws1)
                issue(g + 3, rows1, gs1)

            return carry

        lax.fori_loop(0, npair, body, 0)
        wwait(rows0, ws0)
        wwait(rows1, ws1)

    phase(item_t, hist_iv, hist_o, wid * HIST_PW, HIST_PW // CHB)
    phase(user_t, nbr_iv, nbr_o, wid * NBR_PW, NBR_PW // CHB)

    for idx_hbm, table, out in ((user_i, user_t, user_o),
                                (pos_i, item_t, pos_o),
                                (neg_i, item_t, neg_o)):
        pltpu.sync_copy(idx_hbm.at[wid], one_iv)
        pltpu.async_copy(table.at[one_iv.at[0]],
                         rows0.at[pl.ds(0, SUB)], gs0).wait()
        pltpu.sync_copy(rows0.at[pl.ds(0, SUB)],
                        out.at[pl.ds(wid * ONE_PW, ONE_PW)])


@functools.cache
def _sc_gather():
    return functools.partial(
        pl.kernel,
        out_type=[
            jax.ShapeDtypeStruct((B * HISTP, EDIM), jnp.float32),
            jax.ShapeDtypeStruct((B * NBRSP, EDIM), jnp.float32),
            jax.ShapeDtypeStruct((B, EDIM), jnp.float32),
            jax.ShapeDtypeStruct((B, EDIM), jnp.float32),
            jax.ShapeDtypeStruct((B, EDIM), jnp.float32),
        ],
        mesh=plsc.VectorSubcoreMesh(core_axis_name="c", subcore_axis_name="s"),
        compiler_params=pltpu.CompilerParams(use_tc_tiling_on_sc=False),
        scratch_types=[
            pltpu.VMEM((HIST_PW // SUB, SUB), jnp.int32),
            pltpu.VMEM((NBR_PW // SUB, SUB), jnp.int32),
            pltpu.VMEM((1, SUB), jnp.int32),
            pltpu.VMEM((CHB, EDIM), jnp.float32),
            pltpu.VMEM((CHB, EDIM), jnp.float32),
            pltpu.SemaphoreType.DMA,
            pltpu.SemaphoreType.DMA,
            pltpu.SemaphoreType.DMA,
            pltpu.SemaphoreType.DMA,
        ],
    )(_sc_gather_body)


def _tc_forward_body(hist_ref, histidx_ref, nbr_ref, nbridx_ref,
                     u_ref, pos_ref, neg_ref,
                     ia_wblk, ia_wu, ia_b1, ia_w2sel,
                     ua_wblk, ua_wu, ua_b1, ua_w2sel,
                     rep, scat,
                     fusew, fuseb, selfw, selfb, ul1w, ul1b, ul2w, ul2b,
                     il1w, il1b, il2w, il2b, rp1w, rp1b, rp2w, rp2b,
                     rp3wt, rp3b, pos_out, neg_out):
    u = u_ref[...]

    def attn(xp3, idx4, wblk, wu, b1, w2sel, R):
        pre = jnp.dot(u, wu) + b1
        pre128 = jnp.concatenate([pre] * 4, axis=1)
        t2 = jnp.dot(xp3.reshape(BB * R, 128), wblk)
        h3 = jnp.maximum(t2.reshape(BB, R, 128) + pre128[:, None, :], 0.0)
        s4 = jnp.dot(h3.reshape(BB * R, 128), w2sel).reshape(BB, R, 4)
        e4 = jnp.exp(s4) * (idx4 != 0).astype(jnp.float32)
        z = jnp.sum(jnp.sum(e4, axis=1), axis=1, keepdims=True)
        a128 = jnp.dot(e4.reshape(BB * R, 4), rep[...]).reshape(BB, R, 128)
        w = jnp.sum(a128 * xp3, axis=1)
        return jnp.dot(w, scat[...]) / jnp.maximum(z, 1e-35)

    h_item = attn(hist_ref[...], histidx_ref[...], ia_wblk[...],
                  ia_wu[...], ia_b1[...], ia_w2sel[...], RH)
    h_soc = attn(nbr_ref[...], nbridx_ref[...], ua_wblk[...],
                 ua_wu[...], ua_b1[...], ua_w2sel[...], RN)

    h = jnp.maximum(
        jnp.dot(jnp.concatenate([h_item, h_soc], axis=1), fusew[...]) + fuseb[...],
        0.0)
    hu = jnp.dot(jnp.concatenate([h, u], axis=1), selfw[...]) + selfb[...]
    hu = jnp.dot(jnp.maximum(jnp.dot(hu, ul1w[...]) + ul1b[...], 0.0),
                 ul2w[...]) + ul2b[...]

    def item_mlp(x):
        return jnp.dot(jnp.maximum(jnp.dot(x, il1w[...]) + il1b[...], 0.0),
                       il2w[...]) + il2b[...]

    ph = item_mlp(pos_ref[...])
    nh = item_mlp(neg_ref[...])

    def rp(x):
        x = jnp.maximum(jnp.dot(x, rp1w[...]) + rp1b[...], 0.0)
        x = jnp.maximum(jnp.dot(x, rp2w[...]) + rp2b[...], 0.0)
        return jnp.sum(x * rp3wt[...], axis=1, keepdims=True) + rp3b[0, 0]

    pos_out[...] = rp(jnp.concatenate([hu, ph], axis=1))
    neg_out[...] = rp(jnp.concatenate([hu, nh], axis=1))


def _tc_forward(histp, histidx4, nbrp, nbridx4, u_rows, pos_rows, neg_rows,
                weights):
    grid = (B // BB,)
    data_specs = [
        pl.BlockSpec((BB, RH, 128), lambda i: (i, 0, 0)),
        pl.BlockSpec((BB, RH, 4), lambda i: (i, 0, 0)),
        pl.BlockSpec((BB, RN, 128), lambda i: (i, 0, 0)),
        pl.BlockSpec((BB, RN, 4), lambda i: (i, 0, 0)),
        pl.BlockSpec((BB, EDIM), lambda i: (i, 0)),
        pl.BlockSpec((BB, EDIM), lambda i: (i, 0)),
        pl.BlockSpec((BB, EDIM), lambda i: (i, 0)),
    ]
    w_specs = [pl.BlockSpec(w.shape, lambda i: (0,) * w.ndim) for w in weights]
    out_specs = [pl.BlockSpec((BB, 1), lambda i: (i, 0))] * 2
    return pl.pallas_call(
        _tc_forward_body,
        grid=grid,
        in_specs=data_specs + w_specs,
        out_specs=out_specs,
        out_shape=[jax.ShapeDtypeStruct((B, 1), jnp.float32)] * 2,
    )(histp, histidx4, nbrp, nbridx4, u_rows, pos_rows, neg_rows, *weights)


def kernel(user, user_hist, user_nbrs, pos_item, neg_item, params):
    item_t = params['item_embs']
    user_t = params['user_embs']

    hist_pad = jnp.pad(user_hist, ((0, 0), (0, HISTP - HIST)))
    nbr_pad = jnp.pad(user_nbrs, ((0, 0), (0, NBRSP - NBRS)))

    hist_i = hist_pad.reshape(NW, HIST_PW // SUB, SUB)
    nbr_i = nbr_pad.reshape(NW, NBR_PW // SUB, SUB)
    user_i = user.reshape(NW, 1, SUB)
    pos_i = pos_item.reshape(NW, 1, SUB)
    neg_i = neg_item.reshape(NW, 1, SUB)

    hist_rows, nbr_rows, u_rows, pos_rows, neg_rows = _sc_gather()(
        item_t, user_t, hist_i, nbr_i, user_i, pos_i, neg_i)

    histp = hist_rows.reshape(B, RH, 128)
    nbrp = nbr_rows.reshape(B, RN, 128)
    histidx4 = hist_pad.reshape(B, RH, 4)
    nbridx4 = nbr_pad.reshape(B, RN, 4)

    p = params
    eye4 = jnp.eye(4, dtype=jnp.float32)

    def row(b):
        return b.reshape(1, -1)

    def attn_weights(p1, p2):
        w1, b1 = p1
        w2, _ = p2
        wblk = jnp.kron(eye4, w1[:EDIM, :])
        wu = w1[EDIM:, :]
        w2sel = jnp.kron(eye4, w2)
        return wblk, wu, row(b1), w2sel

    rep = jnp.kron(eye4, jnp.ones((1, EDIM), jnp.float32))
    scat = jnp.kron(jnp.ones((4, 1), jnp.float32),
                    jnp.eye(EDIM, dtype=jnp.float32))

    weights = (
        *attn_weights(p['ia1'], p['ia2']),
        *attn_weights(p['ua1'], p['ua2']),
        rep, scat,
        p['fuse'][0], row(p['fuse'][1]), p['self'][0], row(p['self'][1]),
        p['ul1'][0], row(p['ul1'][1]), p['ul2'][0], row(p['ul2'][1]),
        p['il1'][0], row(p['il1'][1]), p['il2'][0], row(p['il2'][1]),
        p['rp1'][0], row(p['rp1'][1]), p['rp2'][0], row(p['rp2'][1]),
        p['rp3'][0].T, p['rp3'][1].reshape(1, 1),
    )

    return _tc_forward(histp, histidx4, nbrp, nbridx4,
                       u_rows, pos_rows, neg_rows, weights)


# speedup vs baseline: 1.1096x; 1.0001x over previous
import functools

import jax
import jax.numpy as jnp
from jax import lax
from jax.experimental import pallas as pl
from jax.experimental.pallas import tpu as pltpu
from jax.experimental.pallas import tpu_sc as plsc

B = 4096
HIST = 200
NBRS = 50
EDIM = 32
HISTP = 224
NBRSP = 64
RH = HISTP // 4
RN = NBRSP // 4

NW = 32
CHB = 1024

HIST_PW = B * HISTP // NW
NBR_PW = B * NBRSP // NW
ONE_PW = B // NW

BB = 128


def _sc_gather_body(item_t, user_t, hist_i, nbr_i, user_i, pos_i, neg_i,
                    hist_o, nbr_o, user_o, pos_o, neg_o,
                    hist_iv, nbr_iv, one_iv, rows0, rows1,
                    gs0, gs1, ws0, ws1):
    c = lax.axis_index("c")
    s = lax.axis_index("s")
    wid = s * 2 + c

    pltpu.sync_copy(hist_i.at[wid], hist_iv)
    pltpu.sync_copy(nbr_i.at[wid], nbr_iv)

    def phase(table, idx_v, out, base, ngrp):
        def issue(g, buf, sem):
            pltpu.async_copy(table.at[idx_v.at[pl.ds(g * CHB, CHB)]],
                             buf, sem)

        def gwait(buf, sem):
            pltpu.make_async_copy(table.at[pl.ds(0, CHB)], buf, sem).wait()

        def wstart(g, buf, sem):
            pltpu.async_copy(buf, out.at[pl.ds(base + g * CHB, CHB)], sem)

        def wwait(buf, sem):
            pltpu.make_async_copy(buf, out.at[pl.ds(base, CHB)], sem).wait()

        npair = ngrp // 2
        issue(0, rows0, gs0)
        issue(1, rows1, gs1)

        def body(t, carry):
            g = 2 * t
            gwait(rows0, gs0)
            wstart(g, rows0, ws0)
            gwait(rows1, gs1)
            wstart(g + 1, rows1, ws1)

            @pl.when(t + 1 < npair)
            def _():
                wwait(rows0, ws0)
                issue(g + 2, rows0, gs0)
                wwait(rows1, ws1)
                issue(g + 3, rows1, gs1)

            return carry

        lax.fori_loop(0, npair, body, 0)
        wwait(rows0, ws0)
        wwait(rows1, ws1)

    phase(item_t, hist_iv, hist_o, wid * HIST_PW, HIST_PW // CHB)
    phase(user_t, nbr_iv, nbr_o, wid * NBR_PW, NBR_PW // CHB)

    for idx_hbm, table, out in ((user_i, user_t, user_o),
                                (pos_i, item_t, pos_o),
                                (neg_i, item_t, neg_o)):
        pltpu.sync_copy(idx_hbm.at[wid], one_iv)
        pltpu.async_copy(table.at[one_iv],
                         rows0.at[pl.ds(0, ONE_PW)], gs0).wait()
        pltpu.sync_copy(rows0.at[pl.ds(0, ONE_PW)],
                        out.at[pl.ds(wid * ONE_PW, ONE_PW)])


@functools.cache
def _sc_gather():
    return functools.partial(
        pl.kernel,
        out_type=[
            jax.ShapeDtypeStruct((B * HISTP, EDIM), jnp.float32),
            jax.ShapeDtypeStruct((B * NBRSP, EDIM), jnp.float32),
            jax.ShapeDtypeStruct((B, EDIM), jnp.float32),
            jax.ShapeDtypeStruct((B, EDIM), jnp.float32),
            jax.ShapeDtypeStruct((B, EDIM), jnp.float32),
        ],
        mesh=plsc.VectorSubcoreMesh(core_axis_name="c", subcore_axis_name="s"),
        compiler_params=pltpu.CompilerParams(use_tc_tiling_on_sc=False),
        scratch_types=[
            pltpu.VMEM((HIST_PW,), jnp.int32),
            pltpu.VMEM((NBR_PW,), jnp.int32),
            pltpu.VMEM((ONE_PW,), jnp.int32),
            pltpu.VMEM((CHB, EDIM), jnp.float32),
            pltpu.VMEM((CHB, EDIM), jnp.float32),
            pltpu.SemaphoreType.DMA,
            pltpu.SemaphoreType.DMA,
            pltpu.SemaphoreType.DMA,
            pltpu.SemaphoreType.DMA,
        ],
    )(_sc_gather_body)


def _tc_forward_body(hist_ref, histidx_ref, nbr_ref, nbridx_ref,
                     u_ref, pos_ref, neg_ref,
                     ia_wblk, ia_wu, ia_b1, ia_w2sel,
                     ua_wblk, ua_wu, ua_b1, ua_w2sel,
                     rep, scat,
                     fusew, fuseb, selfw, selfb, ul1w, ul1b, ul2w, ul2b,
                     il1w, il1b, il2w, il2b, rp1w, rp1b, rp2w, rp2b,
                     rp3wt, rp3b, pos_out, neg_out):
    u = u_ref[...]

    def attn(xp3, idx4, wblk, wu, b1, w2sel, R):
        pre = jnp.dot(u, wu) + b1
        pre128 = jnp.concatenate([pre] * 4, axis=1)
        t2 = jnp.dot(xp3.reshape(BB * R, 128), wblk)
        h3 = jnp.maximum(t2.reshape(BB, R, 128) + pre128[:, None, :], 0.0)
        s4 = jnp.dot(h3.reshape(BB * R, 128), w2sel).reshape(BB, R, 4)
        e4 = jnp.exp(s4) * (idx4 != 0).astype(jnp.float32)
        z = jnp.sum(jnp.sum(e4, axis=1), axis=1, keepdims=True)
        a128 = jnp.dot(e4.reshape(BB * R, 4), rep[...]).reshape(BB, R, 128)
        w = jnp.sum(a128 * xp3, axis=1)
        return jnp.dot(w, scat[...]) / jnp.maximum(z, 1e-35)

    h_item = attn(hist_ref[...], histidx_ref[...], ia_wblk[...],
                  ia_wu[...], ia_b1[...], ia_w2sel[...], RH)
    h_soc = attn(nbr_ref[...], nbridx_ref[...], ua_wblk[...],
                 ua_wu[...], ua_b1[...], ua_w2sel[...], RN)

    h = jnp.maximum(
        jnp.dot(jnp.concatenate([h_item, h_soc], axis=1), fusew[...]) + fuseb[...],
        0.0)
    hu = jnp.dot(jnp.concatenate([h, u], axis=1), selfw[...]) + selfb[...]
    hu = jnp.dot(jnp.maximum(jnp.dot(hu, ul1w[...]) + ul1b[...], 0.0),
                 ul2w[...]) + ul2b[...]

    def item_mlp(x):
        return jnp.dot(jnp.maximum(jnp.dot(x, il1w[...]) + il1b[...], 0.0),
                       il2w[...]) + il2b[...]

    ph = item_mlp(pos_ref[...])
    nh = item_mlp(neg_ref[...])

    def rp(x):
        x = jnp.maximum(jnp.dot(x, rp1w[...]) + rp1b[...], 0.0)
        x = jnp.maximum(jnp.dot(x, rp2w[...]) + rp2b[...], 0.0)
        return jnp.sum(x * rp3wt[...], axis=1, keepdims=True) + rp3b[0, 0]

    pos_out[...] = rp(jnp.concatenate([hu, ph], axis=1))
    neg_out[...] = rp(jnp.concatenate([hu, nh], axis=1))


def _tc_forward(histp, histidx4, nbrp, nbridx4, u_rows, pos_rows, neg_rows,
                weights):
    grid = (B // BB,)
    data_specs = [
        pl.BlockSpec((BB, RH, 128), lambda i: (i, 0, 0)),
        pl.BlockSpec((BB, RH, 4), lambda i: (i, 0, 0)),
        pl.BlockSpec((BB, RN, 128), lambda i: (i, 0, 0)),
        pl.BlockSpec((BB, RN, 4), lambda i: (i, 0, 0)),
        pl.BlockSpec((BB, EDIM), lambda i: (i, 0)),
        pl.BlockSpec((BB, EDIM), lambda i: (i, 0)),
        pl.BlockSpec((BB, EDIM), lambda i: (i, 0)),
    ]
    w_specs = [pl.BlockSpec(w.shape, lambda i: (0,) * w.ndim) for w in weights]
    out_specs = [pl.BlockSpec((BB, 1), lambda i: (i, 0))] * 2
    return pl.pallas_call(
        _tc_forward_body,
        grid=grid,
        in_specs=data_specs + w_specs,
        out_specs=out_specs,
        out_shape=[jax.ShapeDtypeStruct((B, 1), jnp.float32)] * 2,
    )(histp, histidx4, nbrp, nbridx4, u_rows, pos_rows, neg_rows, *weights)


def kernel(user, user_hist, user_nbrs, pos_item, neg_item, params):
    item_t = params['item_embs']
    user_t = params['user_embs']

    hist_pad = jnp.pad(user_hist, ((0, 0), (0, HISTP - HIST)))
    nbr_pad = jnp.pad(user_nbrs, ((0, 0), (0, NBRSP - NBRS)))

    hist_i = hist_pad.reshape(NW, HIST_PW)
    nbr_i = nbr_pad.reshape(NW, NBR_PW)
    user_i = user.reshape(NW, ONE_PW)
    pos_i = pos_item.reshape(NW, ONE_PW)
    neg_i = neg_item.reshape(NW, ONE_PW)

    hist_rows, nbr_rows, u_rows, pos_rows, neg_rows = _sc_gather()(
        item_t, user_t, hist_i, nbr_i, user_i, pos_i, neg_i)

    histp = hist_rows.reshape(B, RH, 128)
    nbrp = nbr_rows.reshape(B, RN, 128)
    histidx4 = hist_pad.reshape(B, RH, 4)
    nbridx4 = nbr_pad.reshape(B, RN, 4)

    p = params
    eye4 = jnp.eye(4, dtype=jnp.float32)

    def row(b):
        return b.reshape(1, -1)

    def attn_weights(p1, p2):
        w1, b1 = p1
        w2, _ = p2
        wblk = jnp.kron(eye4, w1[:EDIM, :])
        wu = w1[EDIM:, :]
        w2sel = jnp.kron(eye4, w2)
        return wblk, wu, row(b1), w2sel

    rep = jnp.kron(eye4, jnp.ones((1, EDIM), jnp.float32))
    scat = jnp.kron(jnp.ones((4, 1), jnp.float32),
                    jnp.eye(EDIM, dtype=jnp.float32))

    weights = (
        *attn_weights(p['ia1'], p['ia2']),
        *attn_weights(p['ua1'], p['ua2']),
        rep, scat,
        p['fuse'][0], row(p['fuse'][1]), p['self'][0], row(p['self'][1]),
        p['ul1'][0], row(p['ul1'][1]), p['ul2'][0], row(p['ul2'][1]),
        p['il1'][0], row(p['il1'][1]), p['il2'][0], row(p['il2'][1]),
        p['rp1'][0], row(p['rp1'][1]), p['rp2'][0], row(p['rp2'][1]),
        p['rp3'][0].T, p['rp3'][1].reshape(1, 1),
    )

    return _tc_forward(histp, histidx4, nbrp, nbridx4,
                       u_rows, pos_rows, neg_rows, weights)


# speedup vs baseline: 1.6033x; 1.4449x over previous
import functools

import jax
import jax.numpy as jnp
from jax import lax
from jax.experimental import pallas as pl
from jax.experimental.pallas import tpu as pltpu
from jax.experimental.pallas import tpu_sc as plsc

B = 4096
HIST = 200
NBRS = 50
EDIM = 32
HISTP = 200
NBRSP = 64
RH = HISTP // 4
RN = NBRSP // 4

NW = 32
CHB = 1024
CHB_H = 800

HIST_PW = B * HISTP // NW
NBR_PW = B * NBRSP // NW
ONE_PW = B // NW

BB = 128


def _sc_gather_body(item_t, user_t, hist_i, nbr_i, user_i, pos_i, neg_i,
                    hist_o, nbr_o, user_o, pos_o, neg_o,
                    hist_iv, nbr_iv, one_iv, rows0, rows1,
                    gs0, gs1, ws0, ws1):
    c = lax.axis_index("c")
    s = lax.axis_index("s")
    wid = s * 2 + c

    pltpu.sync_copy(hist_i.at[wid], hist_iv)
    pltpu.sync_copy(nbr_i.at[wid], nbr_iv)

    def phase(table, idx_v, out, base, ngrp, chb):
        def issue(g, buf, sem):
            pltpu.async_copy(table.at[idx_v.at[pl.ds(g * chb, chb)]],
                             buf.at[pl.ds(0, chb)], sem)

        def gwait(buf, sem):
            pltpu.make_async_copy(table.at[pl.ds(0, chb)],
                                  buf.at[pl.ds(0, chb)], sem).wait()

        def wstart(g, buf, sem):
            pltpu.async_copy(buf.at[pl.ds(0, chb)],
                             out.at[pl.ds(base + g * chb, chb)], sem)

        def wwait(buf, sem):
            pltpu.make_async_copy(buf.at[pl.ds(0, chb)],
                                  out.at[pl.ds(base, chb)], sem).wait()

        npair = ngrp // 2
        issue(0, rows0, gs0)
        issue(1, rows1, gs1)

        def body(t, carry):
            g = 2 * t
            gwait(rows0, gs0)
            wstart(g, rows0, ws0)
            gwait(rows1, gs1)
            wstart(g + 1, rows1, ws1)

            @pl.when(t + 1 < npair)
            def _():
                wwait(rows0, ws0)
                issue(g + 2, rows0, gs0)
                wwait(rows1, ws1)
                issue(g + 3, rows1, gs1)

            return carry

        lax.fori_loop(0, npair, body, 0)
        wwait(rows0, ws0)
        wwait(rows1, ws1)

    phase(item_t, hist_iv, hist_o, wid * HIST_PW, HIST_PW // CHB_H, CHB_H)
    phase(user_t, nbr_iv, nbr_o, wid * NBR_PW, NBR_PW // CHB, CHB)

    for idx_hbm, table, out in ((user_i, user_t, user_o),
                                (pos_i, item_t, pos_o),
                                (neg_i, item_t, neg_o)):
        pltpu.sync_copy(idx_hbm.at[wid], one_iv)
        pltpu.async_copy(table.at[one_iv],
                         rows0.at[pl.ds(0, ONE_PW)], gs0).wait()
        pltpu.sync_copy(rows0.at[pl.ds(0, ONE_PW)],
                        out.at[pl.ds(wid * ONE_PW, ONE_PW)])


@functools.cache
def _sc_gather():
    return functools.partial(
        pl.kernel,
        out_type=[
            jax.ShapeDtypeStruct((B * HISTP, EDIM), jnp.float32),
            jax.ShapeDtypeStruct((B * NBRSP, EDIM), jnp.float32),
            jax.ShapeDtypeStruct((B, EDIM), jnp.float32),
            jax.ShapeDtypeStruct((B, EDIM), jnp.float32),
            jax.ShapeDtypeStruct((B, EDIM), jnp.float32),
        ],
        mesh=plsc.VectorSubcoreMesh(core_axis_name="c", subcore_axis_name="s"),
        compiler_params=pltpu.CompilerParams(use_tc_tiling_on_sc=False),
        scratch_types=[
            pltpu.VMEM((HIST_PW,), jnp.int32),
            pltpu.VMEM((NBR_PW,), jnp.int32),
            pltpu.VMEM((ONE_PW,), jnp.int32),
            pltpu.VMEM((CHB, EDIM), jnp.float32),
            pltpu.VMEM((CHB, EDIM), jnp.float32),
            pltpu.SemaphoreType.DMA,
            pltpu.SemaphoreType.DMA,
            pltpu.SemaphoreType.DMA,
            pltpu.SemaphoreType.DMA,
        ],
    )(_sc_gather_body)


def _tc_forward_body(hist_ref, histidx_ref, nbr_ref, nbridx_ref,
                     u_ref, pos_ref, neg_ref,
                     ia_wblk, ia_wu, ia_b1, ia_w2sel,
                     ua_wblk, ua_wu, ua_b1, ua_w2sel,
                     rep, scat,
                     fusew, fuseb, selfw, selfb, ul1w, ul1b, ul2w, ul2b,
                     il1w, il1b, il2w, il2b, rp1w, rp1b, rp2w, rp2b,
                     rp3wt, rp3b, pos_out, neg_out):
    u = u_ref[...]

    def attn(xp3, idx4, wblk, wu, b1, w2sel, R):
        pre = jnp.dot(u, wu) + b1
        pre128 = jnp.concatenate([pre] * 4, axis=1)
        t2 = jnp.dot(xp3.reshape(BB * R, 128), wblk)
        h3 = jnp.maximum(t2.reshape(BB, R, 128) + pre128[:, None, :], 0.0)
        s4 = jnp.dot(h3.reshape(BB * R, 128), w2sel).reshape(BB, R, 4)
        e4 = jnp.exp(s4) * (idx4 != 0).astype(jnp.float32)
        z = jnp.sum(jnp.sum(e4, axis=1), axis=1, keepdims=True)
        a128 = jnp.dot(e4.reshape(BB * R, 4), rep[...]).reshape(BB, R, 128)
        w = jnp.sum(a128 * xp3, axis=1)
        return jnp.dot(w, scat[...]) / jnp.maximum(z, 1e-35)

    h_item = attn(hist_ref[...], histidx_ref[...], ia_wblk[...],
                  ia_wu[...], ia_b1[...], ia_w2sel[...], RH)
    h_soc = attn(nbr_ref[...], nbridx_ref[...], ua_wblk[...],
                 ua_wu[...], ua_b1[...], ua_w2sel[...], RN)

    h = jnp.maximum(
        jnp.dot(jnp.concatenate([h_item, h_soc], axis=1), fusew[...]) + fuseb[...],
        0.0)
    hu = jnp.dot(jnp.concatenate([h, u], axis=1), selfw[...]) + selfb[...]
    hu = jnp.dot(jnp.maximum(jnp.dot(hu, ul1w[...]) + ul1b[...], 0.0),
                 ul2w[...]) + ul2b[...]

    def item_mlp(x):
        return jnp.dot(jnp.maximum(jnp.dot(x, il1w[...]) + il1b[...], 0.0),
                       il2w[...]) + il2b[...]

    ph = item_mlp(pos_ref[...])
    nh = item_mlp(neg_ref[...])

    def rp(x):
        x = jnp.maximum(jnp.dot(x, rp1w[...]) + rp1b[...], 0.0)
        x = jnp.maximum(jnp.dot(x, rp2w[...]) + rp2b[...], 0.0)
        return jnp.sum(x * rp3wt[...], axis=1, keepdims=True) + rp3b[0, 0]

    pos_out[...] = rp(jnp.concatenate([hu, ph], axis=1))
    neg_out[...] = rp(jnp.concatenate([hu, nh], axis=1))


def _tc_forward(histp, histidx4, nbrp, nbridx4, u_rows, pos_rows, neg_rows,
                weights):
    grid = (B // BB,)
    data_specs = [
        pl.BlockSpec((BB, RH, 128), lambda i: (i, 0, 0)),
        pl.BlockSpec((BB, RH, 4), lambda i: (i, 0, 0)),
        pl.BlockSpec((BB, RN, 128), lambda i: (i, 0, 0)),
        pl.BlockSpec((BB, RN, 4), lambda i: (i, 0, 0)),
        pl.BlockSpec((BB, EDIM), lambda i: (i, 0)),
        pl.BlockSpec((BB, EDIM), lambda i: (i, 0)),
        pl.BlockSpec((BB, EDIM), lambda i: (i, 0)),
    ]
    w_specs = [pl.BlockSpec(w.shape, lambda i: (0,) * w.ndim) for w in weights]
    out_specs = [pl.BlockSpec((BB, 1), lambda i: (i, 0))] * 2
    return pl.pallas_call(
        _tc_forward_body,
        grid=grid,
        in_specs=data_specs + w_specs,
        out_specs=out_specs,
        out_shape=[jax.ShapeDtypeStruct((B, 1), jnp.float32)] * 2,
    )(histp, histidx4, nbrp, nbridx4, u_rows, pos_rows, neg_rows, *weights)


def kernel(user, user_hist, user_nbrs, pos_item, neg_item, params):
    item_t = params['item_embs']
    user_t = params['user_embs']

    nbr_pad = jnp.pad(user_nbrs, ((0, 0), (0, NBRSP - NBRS)))

    hist_i = user_hist.reshape(NW, HIST_PW)
    nbr_i = nbr_pad.reshape(NW, NBR_PW)
    user_i = user.reshape(NW, ONE_PW)
    pos_i = pos_item.reshape(NW, ONE_PW)
    neg_i = neg_item.reshape(NW, ONE_PW)

    hist_rows, nbr_rows, u_rows, pos_rows, neg_rows = _sc_gather()(
        item_t, user_t, hist_i, nbr_i, user_i, pos_i, neg_i)

    histp = hist_rows.reshape(B, RH, 128)
    nbrp = nbr_rows.reshape(B, RN, 128)
    histidx4 = user_hist.reshape(B, RH, 4)
    nbridx4 = nbr_pad.reshape(B, RN, 4)

    p = params
    eye4 = jnp.eye(4, dtype=jnp.float32)

    def row(b):
        return b.reshape(1, -1)

    def attn_weights(p1, p2):
        w1, b1 = p1
        w2, _ = p2
        wblk = jnp.kron(eye4, w1[:EDIM, :])
        wu = w1[EDIM:, :]
        w2sel = jnp.kron(eye4, w2)
        return wblk, wu, row(b1), w2sel

    rep = jnp.kron(eye4, jnp.ones((1, EDIM), jnp.float32))
    scat = jnp.kron(jnp.ones((4, 1), jnp.float32),
                    jnp.eye(EDIM, dtype=jnp.float32))

    weights = (
        *attn_weights(p['ia1'], p['ia2']),
        *attn_weights(p['ua1'], p['ua2']),
        rep, scat,
        p['fuse'][0], row(p['fuse'][1]), p['self'][0], row(p['self'][1]),
        p['ul1'][0], row(p['ul1'][1]), p['ul2'][0], row(p['ul2'][1]),
        p['il1'][0], row(p['il1'][1]), p['il2'][0], row(p['il2'][1]),
        p['rp1'][0], row(p['rp1'][1]), p['rp2'][0], row(p['rp2'][1]),
        p['rp3'][0].T, p['rp3'][1].reshape(1, 1),
    )

    return _tc_forward(histp, histidx4, nbrp, nbridx4,
                       u_rows, pos_rows, neg_rows, weights)
